# baseline (speedup 1.0000x reference)
import jax
import jax.numpy as jnp
from jax.experimental import pallas as pl
from jax.experimental.pallas import tpu as pltpu

TOPK = 100
BLOCK_B = 4


def _fusion_kernel(feat_ref, text_ref, wd_ref, bd_ref, wsp_ref, bsp_ref,
                   wtext_ref, btext_ref, wout_ref, bout_ref, out_ref,
                   sel_ref):
    nb = feat_ref.shape[0]
    hw = feat_ref.shape[2]
    wdb = wd_ref[:].astype(jnp.bfloat16)

    xbs = []
    rows = []
    for i in range(nb):
        xb = feat_ref[i].astype(jnp.bfloat16)
        xbs.append(xb)
        rows.append(jax.lax.dot_general(
            wdb, xb, (((1,), (0,)), ((), ())),
            preferred_element_type=jnp.float32))
    logits = jnp.concatenate(rows, axis=0) + bd_ref[0, 0]

    iv = jax.lax.bitcast_convert_type(logits, jnp.int32)
    key = iv ^ ((iv >> 31) & jnp.int32(0x7FFFFFFF))

    kk = jnp.int32(TOPK)
    cnt_nn = jnp.sum((key >= 0).astype(jnp.int32), axis=1, keepdims=True)
    t_val = jnp.where(cnt_nn >= kk, jnp.int32(0),
                      jnp.int32(-2147483648))

    def search_step(b, t):
        cand = t | jnp.int32(1 << b)
        cnt = jnp.sum((key >= cand).astype(jnp.int32), axis=1, keepdims=True)
        return jnp.where(cnt >= kk, cand, t)

    text = jax.lax.dot_general(
        wtext_ref[:].astype(jnp.bfloat16), text_ref[:].astype(jnp.bfloat16),
        (((1,), (0,)), ((), ())),
        preferred_element_type=jnp.float32)
    tbias = text + btext_ref[:] + bsp_ref[:]
    wspb = wsp_ref[:].astype(jnp.bfloat16)
    woutb = wout_ref[:].astype(jnp.bfloat16)
    bits = list(range(30, -1, -1))
    per = -(-len(bits) // nb)
    for i in range(nb):
        t1 = jax.lax.dot_general(
            wspb, xbs[i], (((1,), (0,)), ((), ())),
            preferred_element_type=jnp.float32) + tbias
        out_ref[i] = jax.lax.dot_general(
            woutb, t1.astype(jnp.bfloat16), (((1,), (0,)), ((), ())),
            preferred_element_type=jnp.float32) + bout_ref[:]
        for b in bits[i * per:(i + 1) * per]:
            t_val = search_step(b, t_val)

    gt = key > t_val
    eq = key == t_val
    sel_ref[...] = (gt | eq).astype(jnp.float32)
    eq_total = jnp.sum(eq.astype(jnp.int32))

    @pl.when(eq_total != nb)
    def _tie_break():
        need = kk - jnp.sum(gt.astype(jnp.int32), axis=1, keepdims=True)
        idx = jax.lax.broadcasted_iota(jnp.int32, (nb, hw), 1)
        i_val = jnp.zeros((nb, 1), jnp.int32)
        for b in range(9, -1, -1):
            cand = i_val | jnp.int32(1 << b)
            cnt = jnp.sum((eq & (idx <= cand)).astype(jnp.int32),
                          axis=1, keepdims=True)
            i_val = jnp.where(cnt <= need, cand, i_val)
        sel_ref[...] = (gt | (eq & (idx <= i_val))).astype(jnp.float32)

    sel = sel_ref[...]
    for i in range(nb):
        out_ref[i] = jnp.where(sel[i:i + 1, :] != 0.0, out_ref[i],
                               feat_ref[i])


def kernel(feat, text_emb, Wd, bd, W_sp, b_sp, W_text, b_text, W_out, b_out):
    b, c, h, w = feat.shape
    hw = h * w
    e = W_sp.shape[0]
    td = text_emb.shape[0]
    nb = BLOCK_B if b % BLOCK_B == 0 else 1
    featf = feat.reshape(b, c, hw)
    out = pl.pallas_call(
        _fusion_kernel,
        grid=(b // nb,),
        in_specs=[
            pl.BlockSpec((nb, c, hw), lambda i: (i, 0, 0)),
            pl.BlockSpec((td, 1), lambda i: (0, 0)),
            pl.BlockSpec((1, c), lambda i: (0, 0)),
            pl.BlockSpec((1, 1), lambda i: (0, 0)),
            pl.BlockSpec((e, c), lambda i: (0, 0)),
            pl.BlockSpec((e, 1), lambda i: (0, 0)),
            pl.BlockSpec((e, td), lambda i: (0, 0)),
            pl.BlockSpec((e, 1), lambda i: (0, 0)),
            pl.BlockSpec((c, e), lambda i: (0, 0)),
            pl.BlockSpec((c, 1), lambda i: (0, 0)),
        ],
        out_specs=pl.BlockSpec((nb, c, hw), lambda i: (i, 0, 0)),
        out_shape=jax.ShapeDtypeStruct((b, c, hw), jnp.float32),
        scratch_shapes=[pltpu.VMEM((nb, hw), jnp.float32)],
    )(featf, text_emb.reshape(td, 1), Wd.reshape(1, c), bd.reshape(1, 1),
      W_sp, b_sp.reshape(e, 1), W_text, b_text.reshape(e, 1),
      W_out, b_out.reshape(c, 1))
    return out.reshape(b, c, h, w)
